# baseline (speedup 1.0000x reference)
import functools

import jax
import jax.numpy as jnp
from jax import lax
from jax.experimental import pallas as pl
from jax.experimental.pallas import tpu as pltpu
from jax.experimental.pallas import tpu_sc as plsc

R = 16384
D = 2048
NIMG = 16
LANES = 16
DL = D // LANES

_info = plsc.get_sparse_core_info()
NC = _info.num_cores
NS = _info.num_subcores
NW = NC * NS
ROWS_PER_W = R // NW
CHUNK = 4
NCHUNK = ROWS_PER_W // CHUNK
NBUF = 8
LAG = 4


def _sc_body(counts_hbm, box_hbm, pos_hbm, out_hbm,
             pos_v, bufs, cnt_v, ind_meta, scale_meta,
             load_sems, store_sems, pos_sem):
    wid = lax.axis_index("s") * NC + lax.axis_index("c")
    base = wid * ROWS_PER_W

    def load(ch, b):
        rows = box_hbm.at[pl.ds(base + ch * CHUNK, CHUNK), :]
        return pltpu.make_async_copy(rows, bufs.at[b], load_sems.at[b])

    def store(ch, b):
        rows = out_hbm.at[pl.ds(base + ch * CHUNK, CHUNK), :]
        return pltpu.make_async_copy(bufs.at[b], rows, store_sems.at[b])

    for b in range(LAG):
        load(b, b).start()

    pltpu.sync_copy(counts_hbm, cnt_v)
    pos_cp = pltpu.make_async_copy(pos_hbm, pos_v.at[pl.ds(0, NIMG), :],
                                   pos_sem)
    pos_cp.start()

    def zero_body(j, _):
        pos_v[NIMG, pl.ds(j * LANES, LANES)] = jnp.zeros((LANES,), jnp.float32)
        return 0
    lax.fori_loop(0, DL, zero_body, 0)

    c = cnt_v[...]
    cf = c.astype(jnp.float32)
    inv = jnp.where(c > 0, 1.0 / cf, 1.0)
    cs = []
    acc = jnp.int32(0)
    for i in range(NIMG):
        acc = acc + c[i]
        cs.append(acc)

    def meta_body(ch, _):
        rid = base + ch * LANES + jnp.arange(LANES, dtype=jnp.int32)
        ind = jnp.zeros((LANES,), jnp.int32)
        for i in range(NIMG):
            ind = ind + jnp.where(cs[i] <= rid, 1, 0)
        scale = jnp.ones((LANES,), jnp.float32)
        for i in range(NIMG):
            scale = jnp.where(ind == i, inv[i], scale)
        ind_meta[pl.ds(ch * LANES, LANES)] = ind
        scale_meta[pl.ds(ch * LANES, LANES)] = scale
        return 0
    lax.fori_loop(0, ROWS_PER_W // LANES, meta_body, 0)
    pos_cp.wait()

    def group_body(g, _):
        for b in range(NBUF):
            ch = g * NBUF + b

            @pl.when(ch >= LAG)
            def _():
                store(ch - LAG, (b - LAG) % NBUF).wait()

            @pl.when(ch + LAG < NCHUNK)
            def _():
                load(ch + LAG, (b + LAG) % NBUF).start()

            load(ch, b).wait()

            @plsc.parallel_loop(0, CHUNK, 1)
            def row_body(r):
                m = ch * CHUNK + r
                ind = ind_meta[pl.ds(m, LANES)][0]
                s = scale_meta[pl.ds(m, LANES)][0]

                @plsc.parallel_loop(0, DL, 1, unroll=16)
                def col_body(j):
                    sl = pl.ds(j * LANES, LANES)
                    bufs[b, r, sl] = bufs[b, r, sl] * s + pos_v[ind, sl]

            store(ch, b).start()
        return 0
    lax.fori_loop(0, NCHUNK // NBUF, group_body, 0)

    for k in range(LAG):
        ch = NCHUNK - LAG + k
        store(ch, ch % NBUF).wait()


@jax.jit
def _pos_embed(counts, box, pos):
    mesh = plsc.VectorSubcoreMesh(core_axis_name="c", subcore_axis_name="s")
    f = pl.kernel(
        _sc_body,
        out_type=jax.ShapeDtypeStruct((R, D), jnp.float32),
        mesh=mesh,
        scratch_types=[
            pltpu.VMEM((NIMG + 1, D), jnp.float32),
            pltpu.VMEM((NBUF, CHUNK, D), jnp.float32),
            pltpu.VMEM((NIMG,), jnp.int32),
            pltpu.VMEM((ROWS_PER_W + LANES,), jnp.int32),
            pltpu.VMEM((ROWS_PER_W + LANES,), jnp.float32),
            pltpu.SemaphoreType.DMA((NBUF,)),
            pltpu.SemaphoreType.DMA((NBUF,)),
            pltpu.SemaphoreType.DMA,
        ],
    )
    return f(counts, box, pos)


def kernel(eachimg_selected_box_nums, box_features, positions):
    return _pos_embed(eachimg_selected_box_nums, box_features, positions)


# speedup vs baseline: 1.0089x; 1.0089x over previous
import functools

import jax
import jax.numpy as jnp
from jax import lax
from jax.experimental import pallas as pl
from jax.experimental.pallas import tpu as pltpu
from jax.experimental.pallas import tpu_sc as plsc

R = 16384
D = 2048
NIMG = 16
LANES = 16
DL = D // LANES

_info = plsc.get_sparse_core_info()
NC = _info.num_cores
NS = _info.num_subcores
NW = NC * NS
ROWS_PER_W = R // NW
CHUNK = 8
NCHUNK = ROWS_PER_W // CHUNK
NBUF = 4
LAG = 2


def _sc_body(counts_hbm, box_hbm, pos_hbm, out_hbm,
             pos_v, bufs, cnt_v, ind_meta, scale_meta,
             load_sems, store_sems, pos_sem):
    wid = lax.axis_index("s") * NC + lax.axis_index("c")
    base = wid * ROWS_PER_W

    def load(ch, b):
        rows = box_hbm.at[pl.ds(base + ch * CHUNK, CHUNK), :]
        return pltpu.make_async_copy(rows, bufs.at[b], load_sems.at[b])

    def store(ch, b):
        rows = out_hbm.at[pl.ds(base + ch * CHUNK, CHUNK), :]
        return pltpu.make_async_copy(bufs.at[b], rows, store_sems.at[b])

    for b in range(NBUF - 1):
        load(b, b).start()

    pltpu.sync_copy(counts_hbm, cnt_v)
    pos_cp = pltpu.make_async_copy(pos_hbm, pos_v.at[pl.ds(0, NIMG), :],
                                   pos_sem)
    pos_cp.start()

    def zero_body(j, _):
        pos_v[NIMG, pl.ds(j * LANES, LANES)] = jnp.zeros((LANES,), jnp.float32)
        return 0
    lax.fori_loop(0, DL, zero_body, 0)

    c = cnt_v[...]
    cf = c.astype(jnp.float32)
    inv = jnp.where(c > 0, 1.0 / cf, 1.0)
    cs = []
    acc = jnp.int32(0)
    for i in range(NIMG):
        acc = acc + c[i]
        cs.append(acc)

    def meta_body(ch, _):
        rid = base + ch * LANES + jnp.arange(LANES, dtype=jnp.int32)
        ind = jnp.zeros((LANES,), jnp.int32)
        for i in range(NIMG):
            ind = ind + jnp.where(cs[i] <= rid, 1, 0)
        scale = jnp.ones((LANES,), jnp.float32)
        for i in range(NIMG):
            scale = jnp.where(ind == i, inv[i], scale)
        ind_meta[pl.ds(ch * LANES, LANES)] = ind
        scale_meta[pl.ds(ch * LANES, LANES)] = scale
        return 0
    lax.fori_loop(0, ROWS_PER_W // LANES, meta_body, 0)
    pos_cp.wait()

    def group_body(g, _):
        for b in range(NBUF):
            ch = g * NBUF + b

            load(ch, b).wait()

            @plsc.parallel_loop(0, CHUNK, 1)
            def row_body(r):
                m = ch * CHUNK + r
                ind = ind_meta[pl.ds(m, LANES)][0]
                s = scale_meta[pl.ds(m, LANES)][0]

                @plsc.parallel_loop(0, DL, 1, unroll=16)
                def col_body(j):
                    sl = pl.ds(j * LANES, LANES)
                    bufs[b, r, sl] = bufs[b, r, sl] * s + pos_v[ind, sl]

            store(ch, b).start()

            @pl.when(ch >= 1)
            def _():
                store(ch - 1, (b - 1) % NBUF).wait()

            @pl.when(ch + NBUF - 1 < NCHUNK)
            def _():
                load(ch + NBUF - 1, (b - 1) % NBUF).start()
        return 0
    lax.fori_loop(0, NCHUNK // NBUF, group_body, 0)

    store(NCHUNK - 1, (NCHUNK - 1) % NBUF).wait()


@jax.jit
def _pos_embed(counts, box, pos):
    mesh = plsc.VectorSubcoreMesh(core_axis_name="c", subcore_axis_name="s")
    f = pl.kernel(
        _sc_body,
        out_type=jax.ShapeDtypeStruct((R, D), jnp.float32),
        mesh=mesh,
        scratch_types=[
            pltpu.VMEM((NIMG + 1, D), jnp.float32),
            pltpu.VMEM((NBUF, CHUNK, D), jnp.float32),
            pltpu.VMEM((NIMG,), jnp.int32),
            pltpu.VMEM((ROWS_PER_W + LANES,), jnp.int32),
            pltpu.VMEM((ROWS_PER_W + LANES,), jnp.float32),
            pltpu.SemaphoreType.DMA((NBUF,)),
            pltpu.SemaphoreType.DMA((NBUF,)),
            pltpu.SemaphoreType.DMA,
        ],
    )
    return f(counts, box, pos)


def kernel(eachimg_selected_box_nums, box_features, positions):
    return _pos_embed(eachimg_selected_box_nums, box_features, positions)
